# initial kernel scaffold (unmeasured)
import jax
import jax.numpy as jnp
from jax import lax
from jax.experimental import pallas as pl
from jax.experimental.pallas import tpu as pltpu

N_DEV = 4
N_HOPS = 2 * (N_DEV - 1)


def kernel(x, w_mat):
    m, k_per = x.shape
    _, n = w_mat.shape
    rows = m // N_DEV

    def body(x_ref, w_ref, out_ref, send_buf, recv_buf, send_sems, recv_sems):
        my = lax.axis_index("i")
        right = lax.rem(my + 1, N_DEV)

        out_ref[:, :] = jnp.dot(
            x_ref[:, :], w_ref[:, :], preferred_element_type=jnp.float32
        )

        for h in range(N_DEV - 1):
            send_idx = lax.rem(my - h + N_DEV, N_DEV)
            recv_idx = lax.rem(my - h - 1 + N_DEV, N_DEV)
            send_buf[:, :] = out_ref[pl.ds(send_idx * rows, rows), :].astype(
                jnp.bfloat16
            )
            rdma = pltpu.make_async_remote_copy(
                src_ref=send_buf,
                dst_ref=recv_buf.at[h],
                send_sem=send_sems.at[h],
                recv_sem=recv_sems.at[h],
                device_id=(right,),
                device_id_type=pl.DeviceIdType.MESH,
            )
            rdma.start()
            rdma.wait()
            out_ref[pl.ds(recv_idx * rows, rows), :] += recv_buf[h].astype(
                jnp.float32
            )

        own = lax.rem(my + 1, N_DEV)
        y = out_ref[pl.ds(own * rows, rows), :]
        out_ref[pl.ds(own * rows, rows), :] = y * (1.0 / (1.0 + jnp.exp(-y)))

        for h in range(N_DEV - 1):
            send_idx = lax.rem(my + 1 - h + N_DEV, N_DEV)
            recv_idx = lax.rem(my - h + N_DEV, N_DEV)
            s = N_DEV - 1 + h
            send_buf[:, :] = out_ref[pl.ds(send_idx * rows, rows), :].astype(
                jnp.bfloat16
            )
            rdma = pltpu.make_async_remote_copy(
                src_ref=send_buf,
                dst_ref=recv_buf.at[s],
                send_sem=send_sems.at[s],
                recv_sem=recv_sems.at[s],
                device_id=(right,),
                device_id_type=pl.DeviceIdType.MESH,
            )
            rdma.start()
            rdma.wait()
            out_ref[pl.ds(recv_idx * rows, rows), :] = recv_buf[s].astype(
                jnp.float32
            )

    return pl.pallas_call(
        body,
        out_shape=jax.ShapeDtypeStruct((m, n), jnp.float32),
        in_specs=[
            pl.BlockSpec(memory_space=pltpu.VMEM),
            pl.BlockSpec(memory_space=pltpu.VMEM),
        ],
        out_specs=pl.BlockSpec(memory_space=pltpu.VMEM),
        scratch_shapes=[
            pltpu.VMEM((rows, n), jnp.bfloat16),
            pltpu.VMEM((N_HOPS, rows, n), jnp.bfloat16),
            pltpu.SemaphoreType.DMA((N_HOPS,)),
            pltpu.SemaphoreType.DMA((N_HOPS,)),
        ],
        compiler_params=pltpu.CompilerParams(collective_id=0),
    )(x, w_mat)


# baseline (device time: 183766 ns/iter reference)
import jax
import jax.numpy as jnp
from jax import lax
from jax.experimental import pallas as pl
from jax.experimental.pallas import tpu as pltpu

N_DEV = 4
N_HOPS = 2 * (N_DEV - 1)


def kernel(x, w_mat):
    m, k_per = x.shape
    _, n = w_mat.shape
    rows = m // N_DEV

    def body(x_ref, w_ref, out_ref, send_buf, recv_buf, send_sems, recv_sems):
        my = lax.axis_index("i")
        right = lax.rem(my + 1, N_DEV)

        out_ref[:, :] = jnp.dot(
            x_ref[:, :], w_ref[:, :], preferred_element_type=jnp.float32
        )

        for h in range(N_DEV - 1):
            send_idx = lax.rem(my - h + N_DEV, N_DEV)
            recv_idx = lax.rem(my - h - 1 + N_DEV, N_DEV)
            send_buf[:, :] = out_ref[pl.ds(send_idx * rows, rows), :].astype(
                jnp.bfloat16
            )
            rdma = pltpu.make_async_remote_copy(
                src_ref=send_buf,
                dst_ref=recv_buf.at[h],
                send_sem=send_sems.at[h],
                recv_sem=recv_sems.at[h],
                device_id=(right,),
                device_id_type=pl.DeviceIdType.MESH,
            )
            rdma.start()
            rdma.wait()
            out_ref[pl.ds(recv_idx * rows, rows), :] += recv_buf[h].astype(
                jnp.float32
            )

        own = lax.rem(my + 1, N_DEV)
        y = out_ref[pl.ds(own * rows, rows), :]
        out_ref[pl.ds(own * rows, rows), :] = y * (1.0 / (1.0 + jnp.exp(-y)))

        for h in range(N_DEV - 1):
            send_idx = lax.rem(my + 1 - h + N_DEV, N_DEV)
            recv_idx = lax.rem(my - h + N_DEV, N_DEV)
            s = N_DEV - 1 + h
            send_buf[:, :] = out_ref[pl.ds(send_idx * rows, rows), :].astype(
                jnp.bfloat16
            )
            rdma = pltpu.make_async_remote_copy(
                src_ref=send_buf,
                dst_ref=recv_buf.at[s],
                send_sem=send_sems.at[s],
                recv_sem=recv_sems.at[s],
                device_id=(right,),
                device_id_type=pl.DeviceIdType.MESH,
            )
            rdma.start()
            rdma.wait()
            out_ref[pl.ds(recv_idx * rows, rows), :] = recv_buf[s].astype(
                jnp.float32
            )

    return pl.pallas_call(
        body,
        out_shape=jax.ShapeDtypeStruct((m, n), jnp.float32),
        in_specs=[
            pl.BlockSpec(memory_space=pltpu.VMEM),
            pl.BlockSpec(memory_space=pltpu.VMEM),
        ],
        out_specs=pl.BlockSpec(memory_space=pltpu.VMEM),
        scratch_shapes=[
            pltpu.VMEM((rows, n), jnp.bfloat16),
            pltpu.VMEM((N_HOPS, rows, n), jnp.bfloat16),
            pltpu.SemaphoreType.DMA((N_HOPS,)),
            pltpu.SemaphoreType.DMA((N_HOPS,)),
        ],
    )(x, w_mat)


# device time: 109845 ns/iter; 1.6730x vs baseline; 1.6730x over previous
import jax
import jax.numpy as jnp
from jax import lax
from jax.experimental import pallas as pl
from jax.experimental.pallas import tpu as pltpu

N_DEV = 4
N_HOPS = 2 * (N_DEV - 1)


def kernel(x, w_mat):
    m, k_per = x.shape
    _, n = w_mat.shape
    rows = m // N_DEV
    n2 = n // 2

    f32 = jnp.float32
    bf16 = jnp.bfloat16

    def body(
        x_ref,
        w_ref,
        out_ref,
        acc_ref,
        w_bf,
        sbuf_r,
        sbuf_l,
        rbuf_r,
        rbuf_l,
        ssem_r,
        rsem_r,
        ssem_l,
        rsem_l,
    ):
        my = lax.axis_index("i")
        right = lax.rem(my + 1, N_DEV)
        left = lax.rem(my + 3, N_DEV)

        def idx(d):
            return lax.rem(my + d + N_DEV, N_DEV)

        def gemm_block(c):
            acc_ref[pl.ds(c * rows, rows), :] = jnp.dot(
                x_ref[pl.ds(c * rows, rows), :].astype(bf16),
                w_bf[:, :],
                preferred_element_type=f32,
            )

        def acc_half(c, half):
            col0 = 0 if half == 0 else n2
            return acc_ref[pl.ds(c * rows, rows), pl.ds(col0, n2)]

        def rdma(src, slot_r, slot_l, h):
            r = pltpu.make_async_remote_copy(
                src_ref=src[0],
                dst_ref=rbuf_r.at[slot_r],
                send_sem=ssem_r.at[h],
                recv_sem=rsem_r.at[slot_r],
                device_id=(right,),
                device_id_type=pl.DeviceIdType.MESH,
            )
            l = pltpu.make_async_remote_copy(
                src_ref=src[1],
                dst_ref=rbuf_l.at[slot_l],
                send_sem=ssem_l.at[h],
                recv_sem=rsem_l.at[slot_l],
                device_id=(left,),
                device_id_type=pl.DeviceIdType.MESH,
            )
            r.start()
            l.start()
            return r, l

        w_bf[:, :] = w_ref[:, :].astype(bf16)
        gemm_block(my)
        sbuf_r[:, :] = acc_half(my, 0).astype(bf16)
        sbuf_l[:, :] = acc_half(my, 1).astype(bf16)
        r0, l0 = rdma((sbuf_r, sbuf_l), 0, 0, 0)
        gemm_block(idx(-1))
        gemm_block(idx(1))
        r0.wait()
        l0.wait()

        sbuf_r[:, :] = (rbuf_r[0].astype(f32) + acc_half(idx(-1), 0)).astype(bf16)
        sbuf_l[:, :] = (rbuf_l[0].astype(f32) + acc_half(idx(1), 1)).astype(bf16)
        r1, l1 = rdma((sbuf_r, sbuf_l), 1, 1, 1)
        gemm_block(idx(2))
        r1.wait()
        l1.wait()

        sbuf_r[:, :] = (rbuf_r[1].astype(f32) + acc_half(idx(2), 0)).astype(bf16)
        sbuf_l[:, :] = (rbuf_l[1].astype(f32) + acc_half(idx(2), 1)).astype(bf16)
        r2, l2 = rdma((sbuf_r, sbuf_l), 2, 2, 2)
        r2.wait()
        l2.wait()

        y_r = rbuf_r[2].astype(f32) + acc_half(idx(1), 0)
        s_r = y_r * (1.0 / (1.0 + jnp.exp(-y_r)))
        out_ref[pl.ds(idx(1) * rows, rows), pl.ds(0, n2)] = s_r
        sbuf_r[:, :] = s_r.astype(bf16)

        y_l = rbuf_l[2].astype(f32) + acc_half(idx(-1), 1)
        s_l = y_l * (1.0 / (1.0 + jnp.exp(-y_l)))
        out_ref[pl.ds(idx(-1) * rows, rows), pl.ds(n2, n2)] = s_l
        sbuf_l[:, :] = s_l.astype(bf16)

        r3, l3 = rdma((sbuf_r, sbuf_l), 3, 3, 3)
        r3.wait()
        l3.wait()

        r4, l4 = rdma((rbuf_r.at[3], rbuf_l.at[3]), 4, 4, 4)
        out_ref[pl.ds(my * rows, rows), pl.ds(0, n2)] = rbuf_r[3].astype(f32)
        out_ref[pl.ds(my * rows, rows), pl.ds(n2, n2)] = rbuf_l[3].astype(f32)
        r4.wait()
        l4.wait()

        r5, l5 = rdma((rbuf_r.at[4], rbuf_l.at[4]), 5, 5, 5)
        out_ref[pl.ds(idx(-1) * rows, rows), pl.ds(0, n2)] = rbuf_r[4].astype(f32)
        out_ref[pl.ds(idx(1) * rows, rows), pl.ds(n2, n2)] = rbuf_l[4].astype(f32)
        r5.wait()
        l5.wait()

        out_ref[pl.ds(idx(2) * rows, rows), pl.ds(0, n2)] = rbuf_r[5].astype(f32)
        out_ref[pl.ds(idx(2) * rows, rows), pl.ds(n2, n2)] = rbuf_l[5].astype(f32)

    return pl.pallas_call(
        body,
        out_shape=jax.ShapeDtypeStruct((m, n), f32),
        in_specs=[
            pl.BlockSpec(memory_space=pltpu.VMEM),
            pl.BlockSpec(memory_space=pltpu.VMEM),
        ],
        out_specs=pl.BlockSpec(memory_space=pltpu.VMEM),
        scratch_shapes=[
            pltpu.VMEM((m, n), f32),
            pltpu.VMEM((k_per, n), bf16),
            pltpu.VMEM((rows, n2), bf16),
            pltpu.VMEM((rows, n2), bf16),
            pltpu.VMEM((N_HOPS, rows, n2), bf16),
            pltpu.VMEM((N_HOPS, rows, n2), bf16),
            pltpu.SemaphoreType.DMA((N_HOPS,)),
            pltpu.SemaphoreType.DMA((N_HOPS,)),
            pltpu.SemaphoreType.DMA((N_HOPS,)),
            pltpu.SemaphoreType.DMA((N_HOPS,)),
        ],
        compiler_params=pltpu.CompilerParams(
            vmem_limit_bytes=100 * 1024 * 1024,
        ),
    )(x, w_mat)


# device time: 98649 ns/iter; 1.8628x vs baseline; 1.1135x over previous
import jax
import jax.numpy as jnp
from jax import lax
from jax.experimental import pallas as pl
from jax.experimental.pallas import tpu as pltpu

N_DEV = 4
N_HOPS = 2 * (N_DEV - 1)
SUB = 2


def kernel(x, w_mat):
    m, k_per = x.shape
    _, n = w_mat.shape
    rows = m // N_DEV
    n2 = n // 2
    nsub = n2 // SUB

    f32 = jnp.float32
    bf16 = jnp.bfloat16

    def body(
        x_ref,
        w_ref,
        out_ref,
        acc_ref,
        w_bf,
        sbuf_r,
        sbuf_l,
        rbuf_r,
        rbuf_l,
        ssem_r,
        rsem_r,
        ssem_l,
        rsem_l,
    ):
        my = lax.axis_index("i")
        right = lax.rem(my + 1, N_DEV)
        left = lax.rem(my + 3, N_DEV)

        def idx(d):
            return lax.rem(my + d + N_DEV, N_DEV)

        def gemm_block(c):
            acc_ref[pl.ds(c * rows, rows), :] = jnp.dot(
                x_ref[pl.ds(c * rows, rows), :].astype(bf16),
                w_bf[:, :],
                preferred_element_type=f32,
            )

        def acc_sub(c, half, s):
            return acc_ref[pl.ds(c * rows, rows), pl.ds(half * n2 + s * nsub, nsub)]

        R = (rbuf_r, ssem_r, rsem_r, right)
        L = (rbuf_l, ssem_l, rsem_l, left)

        def mk(dir_, src, h, s):
            rbuf, ssem, rsem, dev = dir_
            slot = h * SUB + s
            return pltpu.make_async_remote_copy(
                src_ref=src,
                dst_ref=rbuf.at[slot],
                send_sem=ssem.at[slot],
                recv_sem=rsem.at[slot],
                device_id=(dev,),
                device_id_type=pl.DeviceIdType.MESH,
            )

        ops = {}

        def start(key, dir_, src, h, s):
            op = mk(dir_, src, h, s)
            op.start()
            ops[key, h, s] = op

        w_bf[:, :] = w_ref[:, :].astype(bf16)
        gemm_block(my)
        for s in range(SUB):
            sbuf_r[s, :, :] = acc_sub(my, 0, s).astype(bf16)
            start("R", R, sbuf_r.at[s], 0, s)
            sbuf_l[s, :, :] = acc_sub(my, 1, s).astype(bf16)
            start("L", L, sbuf_l.at[s], 0, s)
        gemm_block(idx(-1))
        gemm_block(idx(1))
        gemm_block(idx(2))

        for h, cR, cL in ((1, idx(-1), idx(1)), (2, idx(2), idx(2))):
            for s in range(SUB):
                ops["R", h - 1, s].wait_recv()
                ops["R", h - 1, s].wait_send()
                sbuf_r[s, :, :] = (
                    rbuf_r[(h - 1) * SUB + s].astype(f32) + acc_sub(cR, 0, s)
                ).astype(bf16)
                start("R", R, sbuf_r.at[s], h, s)
                ops["L", h - 1, s].wait_recv()
                ops["L", h - 1, s].wait_send()
                sbuf_l[s, :, :] = (
                    rbuf_l[(h - 1) * SUB + s].astype(f32) + acc_sub(cL, 1, s)
                ).astype(bf16)
                start("L", L, sbuf_l.at[s], h, s)

        for s in range(SUB):
            ops["R", 2, s].wait_recv()
            y = rbuf_r[2 * SUB + s].astype(f32) + acc_sub(idx(1), 0, s)
            sil = y * (1.0 / (1.0 + jnp.exp(-y)))
            out_ref[pl.ds(idx(1) * rows, rows), pl.ds(s * nsub, nsub)] = sil
            ops["R", 2, s].wait_send()
            sbuf_r[s, :, :] = sil.astype(bf16)
            start("R", R, sbuf_r.at[s], 3, s)

            ops["L", 2, s].wait_recv()
            y = rbuf_l[2 * SUB + s].astype(f32) + acc_sub(idx(-1), 1, s)
            sil = y * (1.0 / (1.0 + jnp.exp(-y)))
            out_ref[pl.ds(idx(-1) * rows, rows), pl.ds(n2 + s * nsub, nsub)] = sil
            ops["L", 2, s].wait_send()
            sbuf_l[s, :, :] = sil.astype(bf16)
            start("L", L, sbuf_l.at[s], 3, s)

        for h, dR, dL in ((4, 0, 0), (5, -1, 1)):
            for s in range(SUB):
                slot = (h - 1) * SUB + s
                ops["R", h - 1, s].wait_recv()
                start("R", R, rbuf_r.at[slot], h, s)
                out_ref[pl.ds(idx(dR) * rows, rows), pl.ds(s * nsub, nsub)] = (
                    rbuf_r[slot].astype(f32)
                )
                ops["L", h - 1, s].wait_recv()
                start("L", L, rbuf_l.at[slot], h, s)
                out_ref[pl.ds(idx(dL) * rows, rows), pl.ds(n2 + s * nsub, nsub)] = (
                    rbuf_l[slot].astype(f32)
                )

        for s in range(SUB):
            slot = 5 * SUB + s
            ops["R", 5, s].wait_recv()
            out_ref[pl.ds(idx(-2) * rows, rows), pl.ds(s * nsub, nsub)] = (
                rbuf_r[slot].astype(f32)
            )
            ops["L", 5, s].wait_recv()
            out_ref[pl.ds(idx(2) * rows, rows), pl.ds(n2 + s * nsub, nsub)] = (
                rbuf_l[slot].astype(f32)
            )
        for h in (3, 4, 5):
            for s in range(SUB):
                ops["R", h, s].wait_send()
                ops["L", h, s].wait_send()

    return pl.pallas_call(
        body,
        out_shape=jax.ShapeDtypeStruct((m, n), f32),
        in_specs=[
            pl.BlockSpec(memory_space=pltpu.VMEM),
            pl.BlockSpec(memory_space=pltpu.VMEM),
        ],
        out_specs=pl.BlockSpec(memory_space=pltpu.VMEM),
        scratch_shapes=[
            pltpu.VMEM((m, n), f32),
            pltpu.VMEM((k_per, n), bf16),
            pltpu.VMEM((SUB, rows, nsub), bf16),
            pltpu.VMEM((SUB, rows, nsub), bf16),
            pltpu.VMEM((N_HOPS * SUB, rows, nsub), bf16),
            pltpu.VMEM((N_HOPS * SUB, rows, nsub), bf16),
            pltpu.SemaphoreType.DMA((N_HOPS * SUB,)),
            pltpu.SemaphoreType.DMA((N_HOPS * SUB,)),
            pltpu.SemaphoreType.DMA((N_HOPS * SUB,)),
            pltpu.SemaphoreType.DMA((N_HOPS * SUB,)),
        ],
        compiler_params=pltpu.CompilerParams(
            vmem_limit_bytes=100 * 1024 * 1024,
        ),
    )(x, w_mat)


# device time: 98461 ns/iter; 1.8664x vs baseline; 1.0019x over previous
import jax
import jax.numpy as jnp
from jax import lax
from jax.experimental import pallas as pl
from jax.experimental.pallas import tpu as pltpu

N_DEV = 4
N_HOPS = 2 * (N_DEV - 1)
SUB = 4


def kernel(x, w_mat):
    m, k_per = x.shape
    _, n = w_mat.shape
    rows = m // N_DEV
    n2 = n // 2
    nsub = n2 // SUB

    f32 = jnp.float32
    bf16 = jnp.bfloat16

    def body(
        x_ref,
        w_ref,
        out_ref,
        acc_ref,
        w_bf,
        sbuf_r,
        sbuf_l,
        rbuf_r,
        rbuf_l,
        ssem_r,
        rsem_r,
        ssem_l,
        rsem_l,
    ):
        my = lax.axis_index("i")
        right = lax.rem(my + 1, N_DEV)
        left = lax.rem(my + 3, N_DEV)

        def idx(d):
            return lax.rem(my + d + N_DEV, N_DEV)

        def gemm_block(c):
            acc_ref[pl.ds(c * rows, rows), :] = jnp.dot(
                x_ref[pl.ds(c * rows, rows), :].astype(bf16),
                w_bf[:, :],
                preferred_element_type=f32,
            )

        def acc_sub(c, half, s):
            return acc_ref[pl.ds(c * rows, rows), pl.ds(half * n2 + s * nsub, nsub)]

        R = (rbuf_r, ssem_r, rsem_r, right)
        L = (rbuf_l, ssem_l, rsem_l, left)

        def mk(dir_, src, h, s):
            rbuf, ssem, rsem, dev = dir_
            slot = h * SUB + s
            return pltpu.make_async_remote_copy(
                src_ref=src,
                dst_ref=rbuf.at[slot],
                send_sem=ssem.at[slot],
                recv_sem=rsem.at[slot],
                device_id=(dev,),
                device_id_type=pl.DeviceIdType.MESH,
            )

        ops = {}

        def start(key, dir_, src, h, s):
            op = mk(dir_, src, h, s)
            op.start()
            ops[key, h, s] = op

        w_bf[:, :] = w_ref[:, :].astype(bf16)
        own_partial = jnp.dot(
            x_ref[pl.ds(my * rows, rows), :].astype(bf16),
            w_bf[:, :],
            preferred_element_type=f32,
        ).astype(bf16)
        for s in range(SUB):
            sbuf_r[s, :, :] = own_partial[:, s * nsub : (s + 1) * nsub]
            start("R", R, sbuf_r.at[s], 0, s)
            sbuf_l[s, :, :] = own_partial[:, n2 + s * nsub : n2 + (s + 1) * nsub]
            start("L", L, sbuf_l.at[s], 0, s)
        gemm_block(idx(-1))
        gemm_block(idx(1))
        gemm_block(idx(2))

        for h, cR, cL in ((1, idx(-1), idx(1)), (2, idx(2), idx(2))):
            for s in range(SUB):
                ops["R", h - 1, s].wait_recv()
                ops["R", h - 1, s].wait_send()
                sbuf_r[s, :, :] = (
                    rbuf_r[(h - 1) * SUB + s].astype(f32) + acc_sub(cR, 0, s)
                ).astype(bf16)
                start("R", R, sbuf_r.at[s], h, s)
                ops["L", h - 1, s].wait_recv()
                ops["L", h - 1, s].wait_send()
                sbuf_l[s, :, :] = (
                    rbuf_l[(h - 1) * SUB + s].astype(f32) + acc_sub(cL, 1, s)
                ).astype(bf16)
                start("L", L, sbuf_l.at[s], h, s)

        for s in range(SUB):
            ops["R", 2, s].wait_recv()
            y = rbuf_r[2 * SUB + s].astype(f32) + acc_sub(idx(1), 0, s)
            sil = y * (1.0 / (1.0 + jnp.exp(-y)))
            out_ref[pl.ds(idx(1) * rows, rows), pl.ds(s * nsub, nsub)] = sil
            ops["R", 2, s].wait_send()
            sbuf_r[s, :, :] = sil.astype(bf16)
            start("R", R, sbuf_r.at[s], 3, s)

            ops["L", 2, s].wait_recv()
            y = rbuf_l[2 * SUB + s].astype(f32) + acc_sub(idx(-1), 1, s)
            sil = y * (1.0 / (1.0 + jnp.exp(-y)))
            out_ref[pl.ds(idx(-1) * rows, rows), pl.ds(n2 + s * nsub, nsub)] = sil
            ops["L", 2, s].wait_send()
            sbuf_l[s, :, :] = sil.astype(bf16)
            start("L", L, sbuf_l.at[s], 3, s)

        for h, dR, dL in ((4, 0, 0), (5, -1, 1)):
            for s in range(SUB):
                slot = (h - 1) * SUB + s
                ops["R", h - 1, s].wait_recv()
                start("R", R, rbuf_r.at[slot], h, s)
                out_ref[pl.ds(idx(dR) * rows, rows), pl.ds(s * nsub, nsub)] = (
                    rbuf_r[slot].astype(f32)
                )
                ops["L", h - 1, s].wait_recv()
                start("L", L, rbuf_l.at[slot], h, s)
                out_ref[pl.ds(idx(dL) * rows, rows), pl.ds(n2 + s * nsub, nsub)] = (
                    rbuf_l[slot].astype(f32)
                )

        for s in range(SUB):
            slot = 5 * SUB + s
            ops["R", 5, s].wait_recv()
            out_ref[pl.ds(idx(-2) * rows, rows), pl.ds(s * nsub, nsub)] = (
                rbuf_r[slot].astype(f32)
            )
            ops["L", 5, s].wait_recv()
            out_ref[pl.ds(idx(2) * rows, rows), pl.ds(n2 + s * nsub, nsub)] = (
                rbuf_l[slot].astype(f32)
            )
        for h in (3, 4, 5):
            for s in range(SUB):
                ops["R", h, s].wait_send()
                ops["L", h, s].wait_send()

    return pl.pallas_call(
        body,
        out_shape=jax.ShapeDtypeStruct((m, n), f32),
        in_specs=[
            pl.BlockSpec(memory_space=pltpu.VMEM),
            pl.BlockSpec(memory_space=pltpu.VMEM),
        ],
        out_specs=pl.BlockSpec(memory_space=pltpu.VMEM),
        scratch_shapes=[
            pltpu.VMEM((m, n), f32),
            pltpu.VMEM((k_per, n), bf16),
            pltpu.VMEM((SUB, rows, nsub), bf16),
            pltpu.VMEM((SUB, rows, nsub), bf16),
            pltpu.VMEM((N_HOPS * SUB, rows, nsub), bf16),
            pltpu.VMEM((N_HOPS * SUB, rows, nsub), bf16),
            pltpu.SemaphoreType.DMA((N_HOPS * SUB,)),
            pltpu.SemaphoreType.DMA((N_HOPS * SUB,)),
            pltpu.SemaphoreType.DMA((N_HOPS * SUB,)),
            pltpu.SemaphoreType.DMA((N_HOPS * SUB,)),
        ],
        compiler_params=pltpu.CompilerParams(
            vmem_limit_bytes=100 * 1024 * 1024,
        ),
    )(x, w_mat)


# device time: 88704 ns/iter; 2.0717x vs baseline; 1.1100x over previous
import jax
import jax.numpy as jnp
from jax import lax
from jax.experimental import pallas as pl
from jax.experimental.pallas import tpu as pltpu

N_DEV = 4
N_HOPS = 2 * (N_DEV - 1)
SUB = 4


def kernel(x, w_mat):
    m, k_per = x.shape
    _, n = w_mat.shape
    rows = m // N_DEV
    n2 = n // 2
    nsub = n2 // SUB

    f32 = jnp.float32
    bf16 = jnp.bfloat16

    def body(
        x_ref,
        w_ref,
        out_ref,
        acc_ref,
        w_bf,
        sbuf_r,
        sbuf_l,
        rbuf_r,
        rbuf_l,
        ssem_r,
        rsem_r,
        ssem_l,
        rsem_l,
        osem_r,
        osem_l,
    ):
        my = lax.axis_index("i")
        right = lax.rem(my + 1, N_DEV)
        left = lax.rem(my + 3, N_DEV)

        def idx(d):
            return lax.rem(my + d + N_DEV, N_DEV)

        def gemm_block(c):
            acc_ref[pl.ds(c * rows, rows), :] = jnp.dot(
                x_ref[pl.ds(c * rows, rows), :].astype(bf16),
                w_bf[:, :],
                preferred_element_type=f32,
            )

        def acc_sub(c, half, s):
            return acc_ref[pl.ds(c * rows, rows), pl.ds(half * n2 + s * nsub, nsub)]

        R = (rbuf_r, ssem_r, rsem_r, right)
        L = (rbuf_l, ssem_l, rsem_l, left)

        def mk(dir_, src, h, s):
            rbuf, ssem, rsem, dev = dir_
            slot = h * SUB + s
            return pltpu.make_async_remote_copy(
                src_ref=src,
                dst_ref=rbuf.at[slot],
                send_sem=ssem.at[slot],
                recv_sem=rsem.at[slot],
                device_id=(dev,),
                device_id_type=pl.DeviceIdType.MESH,
            )

        ops = {}
        stores = []

        def start(key, dir_, src, h, s):
            op = mk(dir_, src, h, s)
            op.start()
            ops[key, h, s] = op

        def store_out(src, row_c, half, k, s, osem):
            cp = pltpu.make_async_copy(
                src,
                out_ref.at[
                    pl.ds(row_c * rows, rows),
                    pl.ds(half * n2 + s * nsub, nsub),
                ],
                osem.at[k * SUB + s],
            )
            cp.start()
            stores.append(cp)

        w_bf[:, :] = w_ref[:, :].astype(bf16)
        own_partial = jnp.dot(
            x_ref[pl.ds(my * rows, rows), :].astype(bf16),
            w_bf[:, :],
            preferred_element_type=f32,
        ).astype(bf16)
        for s in range(SUB):
            sbuf_r[s, :, :] = own_partial[:, s * nsub : (s + 1) * nsub]
            start("R", R, sbuf_r.at[s], 0, s)
            sbuf_l[s, :, :] = own_partial[:, n2 + s * nsub : n2 + (s + 1) * nsub]
            start("L", L, sbuf_l.at[s], 0, s)
        gemm_block(idx(-1))
        gemm_block(idx(1))
        gemm_block(idx(2))

        for h, cR, cL in ((1, idx(-1), idx(1)), (2, idx(2), idx(2))):
            for s in range(SUB):
                ops["R", h - 1, s].wait_recv()
                ops["R", h - 1, s].wait_send()
                sbuf_r[s, :, :] = (
                    rbuf_r[(h - 1) * SUB + s].astype(f32) + acc_sub(cR, 0, s)
                ).astype(bf16)
                start("R", R, sbuf_r.at[s], h, s)
                ops["L", h - 1, s].wait_recv()
                ops["L", h - 1, s].wait_send()
                sbuf_l[s, :, :] = (
                    rbuf_l[(h - 1) * SUB + s].astype(f32) + acc_sub(cL, 1, s)
                ).astype(bf16)
                start("L", L, sbuf_l.at[s], h, s)

        for s in range(SUB):
            ops["R", 2, s].wait_recv()
            y = rbuf_r[2 * SUB + s].astype(f32) + acc_sub(idx(1), 0, s)
            sil = (y * (1.0 / (1.0 + jnp.exp(-y)))).astype(bf16)
            ops["R", 2, s].wait_send()
            sbuf_r[s, :, :] = sil
            start("R", R, sbuf_r.at[s], 3, s)
            store_out(sbuf_r.at[s], idx(1), 0, 0, s, osem_r)

            ops["L", 2, s].wait_recv()
            y = rbuf_l[2 * SUB + s].astype(f32) + acc_sub(idx(-1), 1, s)
            sil = (y * (1.0 / (1.0 + jnp.exp(-y)))).astype(bf16)
            ops["L", 2, s].wait_send()
            sbuf_l[s, :, :] = sil
            start("L", L, sbuf_l.at[s], 3, s)
            store_out(sbuf_l.at[s], idx(-1), 1, 0, s, osem_l)

        for h, dR, dL in ((4, 0, 0), (5, -1, 1)):
            for s in range(SUB):
                slot = (h - 1) * SUB + s
                ops["R", h - 1, s].wait_recv()
                start("R", R, rbuf_r.at[slot], h, s)
                store_out(rbuf_r.at[slot], idx(dR), 0, h - 3, s, osem_r)
                ops["L", h - 1, s].wait_recv()
                start("L", L, rbuf_l.at[slot], h, s)
                store_out(rbuf_l.at[slot], idx(dL), 1, h - 3, s, osem_l)

        for s in range(SUB):
            slot = 5 * SUB + s
            ops["R", 5, s].wait_recv()
            store_out(rbuf_r.at[slot], idx(-2), 0, 3, s, osem_r)
            ops["L", 5, s].wait_recv()
            store_out(rbuf_l.at[slot], idx(2), 1, 3, s, osem_l)
        for cp in stores:
            cp.wait()
        for h in (3, 4, 5):
            for s in range(SUB):
                ops["R", h, s].wait_send()
                ops["L", h, s].wait_send()

    return pl.pallas_call(
        body,
        out_shape=jax.ShapeDtypeStruct((m, n), bf16),
        in_specs=[
            pl.BlockSpec(memory_space=pltpu.VMEM),
            pl.BlockSpec(memory_space=pltpu.VMEM),
        ],
        out_specs=pl.BlockSpec(memory_space=pltpu.MemorySpace.HBM),
        scratch_shapes=[
            pltpu.VMEM((m, n), f32),
            pltpu.VMEM((k_per, n), bf16),
            pltpu.VMEM((SUB, rows, nsub), bf16),
            pltpu.VMEM((SUB, rows, nsub), bf16),
            pltpu.VMEM((N_HOPS * SUB, rows, nsub), bf16),
            pltpu.VMEM((N_HOPS * SUB, rows, nsub), bf16),
            pltpu.SemaphoreType.DMA((N_HOPS * SUB,)),
            pltpu.SemaphoreType.DMA((N_HOPS * SUB,)),
            pltpu.SemaphoreType.DMA((N_HOPS * SUB,)),
            pltpu.SemaphoreType.DMA((N_HOPS * SUB,)),
            pltpu.SemaphoreType.DMA((4 * SUB,)),
            pltpu.SemaphoreType.DMA((4 * SUB,)),
        ],
        compiler_params=pltpu.CompilerParams(
            vmem_limit_bytes=100 * 1024 * 1024,
        ),
    )(x, w_mat)


# device time: 85839 ns/iter; 2.1408x vs baseline; 1.0334x over previous
import jax
import jax.numpy as jnp
from jax import lax
from jax.experimental import pallas as pl
from jax.experimental.pallas import tpu as pltpu

N_DEV = 4
N_HOPS = 2 * (N_DEV - 1)
SUB = 4


def kernel(x, w_mat):
    m, k_per = x.shape
    _, n = w_mat.shape
    rows = m // N_DEV
    n2 = n // 2
    nsub = n2 // SUB

    f32 = jnp.float32
    bf16 = jnp.bfloat16

    def body(
        x_ref,
        w_ref,
        out_ref,
        acc_ref,
        w_bf,
        sbuf_r,
        sbuf_l,
        rbuf_r,
        rbuf_l,
        ssem_r,
        rsem_r,
        ssem_l,
        rsem_l,
        osem_r,
        osem_l,
    ):
        my = lax.axis_index("i")
        right = lax.rem(my + 1, N_DEV)
        left = lax.rem(my + 3, N_DEV)

        def idx(d):
            return lax.rem(my + d + N_DEV, N_DEV)

        def gemm_block(c):
            acc_ref[pl.ds(c * rows, rows), :] = jnp.dot(
                x_ref[pl.ds(c * rows, rows), :].astype(bf16),
                w_bf[:, :],
                preferred_element_type=f32,
            )

        def acc_sub(c, half, s):
            return acc_ref[pl.ds(c * rows, rows), pl.ds(half * n2 + s * nsub, nsub)]

        R = (rbuf_r, ssem_r, rsem_r, right)
        L = (rbuf_l, ssem_l, rsem_l, left)

        def mk(dir_, src, h, s):
            rbuf, ssem, rsem, dev = dir_
            slot = h * SUB + s
            return pltpu.make_async_remote_copy(
                src_ref=src,
                dst_ref=rbuf.at[slot],
                send_sem=ssem.at[slot],
                recv_sem=rsem.at[slot],
                device_id=(dev,),
                device_id_type=pl.DeviceIdType.MESH,
            )

        ops = {}
        stores = []

        def start(key, dir_, src, h, s):
            op = mk(dir_, src, h, s)
            op.start()
            ops[key, h, s] = op

        def store_out(src, row_c, half, k, s, osem):
            cp = pltpu.make_async_copy(
                src,
                out_ref.at[
                    pl.ds(row_c * rows, rows),
                    pl.ds(half * n2 + s * nsub, nsub),
                ],
                osem.at[k * SUB + s],
            )
            cp.start()
            stores.append(cp)

        barrier = pltpu.get_barrier_semaphore()
        pl.semaphore_signal(
            barrier, inc=1, device_id=(left,),
            device_id_type=pl.DeviceIdType.MESH,
        )
        pl.semaphore_signal(
            barrier, inc=1, device_id=(right,),
            device_id_type=pl.DeviceIdType.MESH,
        )
        pl.semaphore_wait(barrier, 2)

        w_bf[:, :] = w_ref[:, :].astype(bf16)
        x_own = x_ref[pl.ds(my * rows, rows), :].astype(bf16)
        for s in range(SUB):
            sbuf_r[s, :, :] = jnp.dot(
                x_own, w_bf[:, pl.ds(s * nsub, nsub)],
                preferred_element_type=f32,
            ).astype(bf16)
            start("R", R, sbuf_r.at[s], 0, s)
            sbuf_l[s, :, :] = jnp.dot(
                x_own, w_bf[:, pl.ds(n2 + s * nsub, nsub)],
                preferred_element_type=f32,
            ).astype(bf16)
            start("L", L, sbuf_l.at[s], 0, s)
        gemm_block(idx(-1))
        gemm_block(idx(1))
        gemm_block(idx(2))

        for h, cR, cL in ((1, idx(-1), idx(1)), (2, idx(2), idx(2))):
            for s in range(SUB):
                ops["R", h - 1, s].wait_recv()
                ops["R", h - 1, s].wait_send()
                sbuf_r[s, :, :] = (
                    rbuf_r[(h - 1) * SUB + s].astype(f32) + acc_sub(cR, 0, s)
                ).astype(bf16)
                start("R", R, sbuf_r.at[s], h, s)
                ops["L", h - 1, s].wait_recv()
                ops["L", h - 1, s].wait_send()
                sbuf_l[s, :, :] = (
                    rbuf_l[(h - 1) * SUB + s].astype(f32) + acc_sub(cL, 1, s)
                ).astype(bf16)
                start("L", L, sbuf_l.at[s], h, s)

        for s in range(SUB):
            ops["R", 2, s].wait_recv()
            y = rbuf_r[2 * SUB + s].astype(f32) + acc_sub(idx(1), 0, s)
            sil = (y * (1.0 / (1.0 + jnp.exp(-y)))).astype(bf16)
            ops["R", 2, s].wait_send()
            sbuf_r[s, :, :] = sil
            start("R", R, sbuf_r.at[s], 3, s)
            store_out(sbuf_r.at[s], idx(1), 0, 0, s, osem_r)

            ops["L", 2, s].wait_recv()
            y = rbuf_l[2 * SUB + s].astype(f32) + acc_sub(idx(-1), 1, s)
            sil = (y * (1.0 / (1.0 + jnp.exp(-y)))).astype(bf16)
            ops["L", 2, s].wait_send()
            sbuf_l[s, :, :] = sil
            start("L", L, sbuf_l.at[s], 3, s)
            store_out(sbuf_l.at[s], idx(-1), 1, 0, s, osem_l)

        for h, dR, dL in ((4, 0, 0), (5, -1, 1)):
            for s in range(SUB):
                slot = (h - 1) * SUB + s
                ops["R", h - 1, s].wait_recv()
                start("R", R, rbuf_r.at[slot], h, s)
                store_out(rbuf_r.at[slot], idx(dR), 0, h - 3, s, osem_r)
                ops["L", h - 1, s].wait_recv()
                start("L", L, rbuf_l.at[slot], h, s)
                store_out(rbuf_l.at[slot], idx(dL), 1, h - 3, s, osem_l)

        for s in range(SUB):
            slot = 5 * SUB + s
            ops["R", 5, s].wait_recv()
            store_out(rbuf_r.at[slot], idx(-2), 0, 3, s, osem_r)
            ops["L", 5, s].wait_recv()
            store_out(rbuf_l.at[slot], idx(2), 1, 3, s, osem_l)
        for cp in stores:
            cp.wait()
        for h in (3, 4, 5):
            for s in range(SUB):
                ops["R", h, s].wait_send()
                ops["L", h, s].wait_send()

    return pl.pallas_call(
        body,
        out_shape=jax.ShapeDtypeStruct((m, n), bf16),
        in_specs=[
            pl.BlockSpec(memory_space=pltpu.VMEM),
            pl.BlockSpec(memory_space=pltpu.VMEM),
        ],
        out_specs=pl.BlockSpec(memory_space=pltpu.MemorySpace.HBM),
        scratch_shapes=[
            pltpu.VMEM((m, n), f32),
            pltpu.VMEM((k_per, n), bf16),
            pltpu.VMEM((SUB, rows, nsub), bf16),
            pltpu.VMEM((SUB, rows, nsub), bf16),
            pltpu.VMEM((N_HOPS * SUB, rows, nsub), bf16),
            pltpu.VMEM((N_HOPS * SUB, rows, nsub), bf16),
            pltpu.SemaphoreType.DMA((N_HOPS * SUB,)),
            pltpu.SemaphoreType.DMA((N_HOPS * SUB,)),
            pltpu.SemaphoreType.DMA((N_HOPS * SUB,)),
            pltpu.SemaphoreType.DMA((N_HOPS * SUB,)),
            pltpu.SemaphoreType.DMA((4 * SUB,)),
            pltpu.SemaphoreType.DMA((4 * SUB,)),
        ],
        compiler_params=pltpu.CompilerParams(
            vmem_limit_bytes=100 * 1024 * 1024,
            collective_id=0,
        ),
    )(x, w_mat)
